# Initial kernel scaffold; baseline (speedup 1.0000x reference)
import jax
import jax.numpy as jnp
from jax.experimental import pallas as pl


def kernel(inputs, word_embedding_matrix):
    raise NotImplementedError("write your pallas kernel here")


# speedup vs baseline: 4.1428x; 4.1428x over previous
import functools

import jax
import jax.numpy as jnp
from jax import lax
from jax.experimental import pallas as pl
from jax.experimental.pallas import tpu as pltpu
from jax.experimental.pallas import tpu_sc as plsc

D = 64
B = 4096 * 200
NC = 2
NS = 16
NW = NC * NS
B_PER_W = B // NW
CHUNK = 1024
N_CHUNKS = B_PER_W // CHUNK

_mesh = plsc.VectorSubcoreMesh(core_axis_name="c", subcore_axis_name="s")


@functools.partial(
    pl.kernel,
    mesh=_mesh,
    out_type=jax.ShapeDtypeStruct((B, D), jnp.float32),
    scratch_types=[
        pltpu.VMEM((CHUNK,), jnp.int32),
        pltpu.VMEM((CHUNK, D), jnp.float32),
        pltpu.SemaphoreType.DMA,
    ],
    compiler_params=pltpu.CompilerParams(use_tc_tiling_on_sc=False),
)
def _gather_kernel(idx_hbm, table_hbm, out_hbm, idx_v, rows_v, sem):
    wid = lax.axis_index("s") * NC + lax.axis_index("c")
    base_w = wid * B_PER_W

    def body(i, carry):
        base = base_w + i * CHUNK
        pltpu.sync_copy(idx_hbm.at[pl.ds(base, CHUNK)], idx_v)
        pltpu.async_copy(table_hbm.at[idx_v], rows_v, sem).wait()
        pltpu.sync_copy(rows_v, out_hbm.at[pl.ds(base, CHUNK)])
        return carry

    lax.fori_loop(0, N_CHUNKS, body, 0)


def kernel(inputs, word_embedding_matrix):
    idx = inputs.reshape(-1).astype(jnp.int32)
    out = _gather_kernel(idx, word_embedding_matrix)
    return out.reshape(inputs.shape + (D,))


# speedup vs baseline: 4.2583x; 1.0279x over previous
import functools

import jax
import jax.numpy as jnp
from jax import lax
from jax.experimental import pallas as pl
from jax.experimental.pallas import tpu as pltpu
from jax.experimental.pallas import tpu_sc as plsc

D = 64
B = 4096 * 200
NC = 2
NS = 16
NW = NC * NS
B_PER_W = B // NW
CHUNK = 800
N_CHUNKS = B_PER_W // CHUNK
N_OUTER = N_CHUNKS // 2

_mesh = plsc.VectorSubcoreMesh(core_axis_name="c", subcore_axis_name="s")


@functools.partial(
    pl.kernel,
    mesh=_mesh,
    out_type=jax.ShapeDtypeStruct((B, D), jnp.float32),
    scratch_types=[
        pltpu.VMEM((B_PER_W,), jnp.int32),
        pltpu.VMEM((CHUNK, D), jnp.float32),
        pltpu.VMEM((CHUNK, D), jnp.float32),
        pltpu.SemaphoreType.DMA,
        pltpu.SemaphoreType.DMA,
        pltpu.SemaphoreType.DMA,
        pltpu.SemaphoreType.DMA,
    ],
    compiler_params=pltpu.CompilerParams(use_tc_tiling_on_sc=False),
)
def _gather_kernel(idx_hbm, table_hbm, out_hbm,
                   idx_all, rows0, rows1, sg0, sg1, ss0, ss1):
    wid = lax.axis_index("s") * NC + lax.axis_index("c")
    base_w = wid * B_PER_W

    pltpu.sync_copy(idx_hbm.at[pl.ds(base_w, B_PER_W)], idx_all)

    def start_gather(g, rows, sem):
        pltpu.async_copy(table_hbm.at[idx_all.at[pl.ds(g * CHUNK, CHUNK)]],
                         rows, sem)

    def wait_gather(rows, sem):
        pltpu.make_async_copy(table_hbm.at[idx_all.at[pl.ds(0, CHUNK)]],
                              rows, sem).wait()

    def start_store(g, rows, sem):
        pltpu.async_copy(rows, out_hbm.at[pl.ds(base_w + g * CHUNK, CHUNK)],
                         sem)

    def wait_store(rows, sem):
        pltpu.make_async_copy(rows, out_hbm.at[pl.ds(base_w, CHUNK)],
                              sem).wait()

    start_gather(0, rows0, sg0)
    start_gather(1, rows1, sg1)
    wait_gather(rows0, sg0)
    start_store(0, rows0, ss0)
    wait_store(rows0, ss0)
    start_gather(2, rows0, sg0)
    wait_gather(rows1, sg1)
    start_store(1, rows1, ss1)

    def body(i, carry):
        g = 2 * i
        wait_store(rows1, ss1)
        start_gather(g + 1, rows1, sg1)
        wait_gather(rows0, sg0)
        start_store(g, rows0, ss0)
        wait_store(rows0, ss0)
        start_gather(g + 2, rows0, sg0)
        wait_gather(rows1, sg1)
        start_store(g + 1, rows1, ss1)
        return carry

    lax.fori_loop(1, N_OUTER - 1, body, 0)

    g = N_CHUNKS - 2
    wait_store(rows1, ss1)
    start_gather(g + 1, rows1, sg1)
    wait_gather(rows0, sg0)
    start_store(g, rows0, ss0)
    wait_gather(rows1, sg1)
    start_store(g + 1, rows1, ss1)
    wait_store(rows0, ss0)
    wait_store(rows1, ss1)


def kernel(inputs, word_embedding_matrix):
    idx = inputs.reshape(-1).astype(jnp.int32)
    out = _gather_kernel(idx, word_embedding_matrix)
    return out.reshape(inputs.shape + (D,))


# speedup vs baseline: 4.2586x; 1.0001x over previous
import functools

import jax
import jax.numpy as jnp
from jax import lax
from jax.experimental import pallas as pl
from jax.experimental.pallas import tpu as pltpu
from jax.experimental.pallas import tpu_sc as plsc

D = 64
B = 4096 * 200
NC = 2
NS = 16
NW = NC * NS
B_PER_W = B // NW
CHUNK = 800
N_CHUNKS = B_PER_W // CHUNK
N_OUTER = N_CHUNKS // 2

_mesh = plsc.VectorSubcoreMesh(core_axis_name="c", subcore_axis_name="s")


BATCH = 4096
HIST = 200


@functools.partial(
    pl.kernel,
    mesh=_mesh,
    out_type=jax.ShapeDtypeStruct((BATCH, HIST, D), jnp.float32),
    scratch_types=[
        pltpu.VMEM((B_PER_W,), jnp.int32),
        pltpu.VMEM((CHUNK, D), jnp.float32),
        pltpu.VMEM((CHUNK, D), jnp.float32),
        pltpu.SemaphoreType.DMA,
        pltpu.SemaphoreType.DMA,
        pltpu.SemaphoreType.DMA,
        pltpu.SemaphoreType.DMA,
    ],
    compiler_params=pltpu.CompilerParams(use_tc_tiling_on_sc=False),
)
def _gather_kernel(idx_hbm, table_hbm, out_hbm,
                   idx_all, rows0, rows1, sg0, sg1, ss0, ss1):
    wid = lax.axis_index("s") * NC + lax.axis_index("c")
    base_w = wid * B_PER_W

    pltpu.sync_copy(idx_hbm.at[pl.ds(base_w, B_PER_W)], idx_all)

    def start_gather(g, rows, sem):
        pltpu.async_copy(table_hbm.at[idx_all.at[pl.ds(g * CHUNK, CHUNK)]],
                         rows, sem)

    def wait_gather(rows, sem):
        pltpu.make_async_copy(table_hbm.at[idx_all.at[pl.ds(0, CHUNK)]],
                              rows, sem).wait()

    def start_store(g, rows, sem):
        row0 = (base_w + g * CHUNK) // HIST
        for k in range(CHUNK // HIST):
            pltpu.async_copy(rows.at[pl.ds(k * HIST, HIST)],
                             out_hbm.at[row0 + k], sem)

    def wait_store(rows, sem):
        for k in range(CHUNK // HIST):
            pltpu.make_async_copy(rows.at[pl.ds(k * HIST, HIST)],
                                  out_hbm.at[0], sem).wait()

    start_gather(0, rows0, sg0)
    start_gather(1, rows1, sg1)
    wait_gather(rows0, sg0)
    start_store(0, rows0, ss0)
    wait_store(rows0, ss0)
    start_gather(2, rows0, sg0)
    wait_gather(rows1, sg1)
    start_store(1, rows1, ss1)

    def body(i, carry):
        g = 2 * i
        wait_store(rows1, ss1)
        start_gather(g + 1, rows1, sg1)
        wait_gather(rows0, sg0)
        start_store(g, rows0, ss0)
        wait_store(rows0, ss0)
        start_gather(g + 2, rows0, sg0)
        wait_gather(rows1, sg1)
        start_store(g + 1, rows1, ss1)
        return carry

    lax.fori_loop(1, N_OUTER - 1, body, 0)

    g = N_CHUNKS - 2
    wait_store(rows1, ss1)
    start_gather(g + 1, rows1, sg1)
    wait_gather(rows0, sg0)
    start_store(g, rows0, ss0)
    wait_gather(rows1, sg1)
    start_store(g + 1, rows1, ss1)
    wait_store(rows0, ss0)
    wait_store(rows1, ss1)


def kernel(inputs, word_embedding_matrix):
    idx = inputs.reshape(-1).astype(jnp.int32)
    return _gather_kernel(idx, word_embedding_matrix)


# speedup vs baseline: 7.4772x; 1.7558x over previous
import functools

import jax
import jax.numpy as jnp
from jax import lax
from jax.experimental import pallas as pl
from jax.experimental.pallas import tpu as pltpu
from jax.experimental.pallas import tpu_sc as plsc

D = 64
DP = 128
BATCH = 4096
HIST = 200
NC = 2
NS = 16
NW = NC * NS
ROWS_PER_W = BATCH // NW
N_OUTER = ROWS_PER_W // 2

_mesh = plsc.VectorSubcoreMesh(core_axis_name="c", subcore_axis_name="s")


@functools.partial(
    pl.kernel,
    mesh=_mesh,
    out_type=jax.ShapeDtypeStruct((BATCH, HIST, DP), jnp.float32),
    scratch_types=[
        pltpu.VMEM((ROWS_PER_W, HIST), jnp.int32),
        pltpu.VMEM((HIST, D), jnp.float32),
        pltpu.VMEM((HIST, D), jnp.float32),
        pltpu.SemaphoreType.DMA,
        pltpu.SemaphoreType.DMA,
        pltpu.SemaphoreType.DMA,
        pltpu.SemaphoreType.DMA,
    ],
    compiler_params=pltpu.CompilerParams(use_tc_tiling_on_sc=False),
)
def _gather_kernel(idx_hbm, table_hbm, out_hbm,
                   idx_all, rows0, rows1, sg0, sg1, ss0, ss1):
    wid = lax.axis_index("s") * NC + lax.axis_index("c")
    base_w = wid * ROWS_PER_W

    pltpu.sync_copy(idx_hbm.at[pl.ds(base_w, ROWS_PER_W)], idx_all)

    def start_gather(j, rows, sem):
        pltpu.async_copy(table_hbm.at[idx_all.at[j]], rows, sem)

    def wait_gather(rows, sem):
        pltpu.make_async_copy(table_hbm.at[idx_all.at[0]], rows, sem).wait()

    def start_store(j, rows, sem):
        pltpu.async_copy(rows,
                         out_hbm.at[base_w + j, pl.ds(0, HIST), pl.ds(0, D)],
                         sem)

    def wait_store(rows, sem):
        pltpu.make_async_copy(rows,
                              out_hbm.at[0, pl.ds(0, HIST), pl.ds(0, D)],
                              sem).wait()

    start_gather(0, rows0, sg0)
    start_gather(1, rows1, sg1)
    wait_gather(rows0, sg0)
    start_store(0, rows0, ss0)
    wait_store(rows0, ss0)
    start_gather(2, rows0, sg0)
    wait_gather(rows1, sg1)
    start_store(1, rows1, ss1)

    def body(i, carry):
        j = 2 * i
        wait_store(rows1, ss1)
        start_gather(j + 1, rows1, sg1)
        wait_gather(rows0, sg0)
        start_store(j, rows0, ss0)
        wait_store(rows0, ss0)
        start_gather(j + 2, rows0, sg0)
        wait_gather(rows1, sg1)
        start_store(j + 1, rows1, ss1)
        return carry

    lax.fori_loop(1, N_OUTER - 1, body, 0)

    j = ROWS_PER_W - 2
    wait_store(rows1, ss1)
    start_gather(j + 1, rows1, sg1)
    wait_gather(rows0, sg0)
    start_store(j, rows0, ss0)
    wait_gather(rows1, sg1)
    start_store(j + 1, rows1, ss1)
    wait_store(rows0, ss0)
    wait_store(rows1, ss1)


def kernel(inputs, word_embedding_matrix):
    idx = inputs.astype(jnp.int32)
    out_p = _gather_kernel(idx, word_embedding_matrix)
    return out_p[..., :D]
